# baseline (speedup 1.0000x reference)
import functools

import jax
import jax.numpy as jnp
from jax import lax
from jax.experimental import pallas as pl
from jax.experimental.pallas import tpu as pltpu
from jax.experimental.pallas import tpu_sc as plsc

_L = 16


def _transpose_into(src_v, dst_v, rows, cols, iotas):
    n_r = rows // _L

    @pl.loop(0, cols, unroll=8)
    def _(c):
        c_s = jnp.full((_L,), c, jnp.int32)
        for k in range(n_r):
            vals = plsc.load_gather(src_v, [iotas[k], c_s])
            dst_v[c, pl.ds(k * _L, _L)] = vals


def _make_fmt(V, D):
    info = plsc.get_sparse_core_info()
    nc, ns = info.num_cores, info.num_subcores
    nw = nc * ns
    n_win = V // 128
    n_tail = V - n_win * 128
    base_cnt, extra = divmod(n_win, nw)
    mesh = plsc.VectorSubcoreMesh(core_axis_name="c", subcore_axis_name="s")

    @functools.partial(
        pl.kernel,
        out_type=jax.ShapeDtypeStruct((V, 2 * D), jnp.float32),
        mesh=mesh,
        scratch_types=[
            pltpu.VMEM((2, D, 128), jnp.float32),
            pltpu.VMEM((2, 128, 2 * D), jnp.float32),
            pltpu.VMEM((128, 2 * D), jnp.float32),
            pltpu.SemaphoreType.DMA,
            pltpu.SemaphoreType.DMA,
        ],
        compiler_params=pltpu.CompilerParams(needs_layout_passes=False),
    )
    def k(tableT_hbm, tail_hbm, out_hbm, a_v, t_v, tl_v, rsem, wsem):
        wid = lax.axis_index("s") * nc + lax.axis_index("c")
        cnt = jnp.where(wid < extra, base_cnt + 1, base_cnt)
        iotas = [lax.iota(jnp.int32, _L) + k * _L for k in range(128 // _L)]

        def col0(t):
            return (wid + t * nw) * 128

        pltpu.async_copy(tableT_hbm.at[:, pl.ds(col0(0), 128)], a_v.at[0], rsem)

        @pl.loop(0, cnt)
        def _(t):
            b = t % 2
            pltpu.make_async_copy(
                tableT_hbm.at[:, pl.ds(0, 128)], a_v.at[b], rsem
            ).wait()

            @pl.when(t + 1 < cnt)
            def _():
                pltpu.async_copy(
                    tableT_hbm.at[:, pl.ds(col0(t + 1), 128)], a_v.at[1 - b], rsem
                )

            @pl.when(t > 1)
            def _():
                pltpu.make_async_copy(
                    out_hbm.at[pl.ds(0, 128)], t_v.at[b], wsem
                ).wait()

            _transpose_into(a_v.at[b], t_v.at[b], D, 128, iotas)
            pltpu.async_copy(t_v.at[b], out_hbm.at[pl.ds(col0(t), 128)], wsem)

        @pl.loop(0, jnp.minimum(cnt, 2))
        def _(i):
            pltpu.make_async_copy(out_hbm.at[pl.ds(0, 128)], t_v.at[0], wsem).wait()

        @pl.when(wid == nw - 1)
        def _():
            pltpu.sync_copy(tail_hbm, tl_v)
            pltpu.sync_copy(
                tl_v.at[pl.ds(0, n_tail)], out_hbm.at[pl.ds(n_win * 128, n_tail)]
            )

    return k


def _make_gather(S, NB, D):
    info = plsc.get_sparse_core_info()
    nc, ns = info.num_cores, info.num_subcores
    nw = nc * ns
    assert NB % (128 * nw) == 0 or NB == 128 * nw
    mesh = plsc.VectorSubcoreMesh(core_axis_name="c", subcore_axis_name="s")

    @functools.partial(
        pl.kernel,
        out_type=jax.ShapeDtypeStruct((S, D, NB), jnp.float32),
        mesh=mesh,
        scratch_types=[
            pltpu.VMEM((S, 128), jnp.int32),
            pltpu.VMEM((2, 128, 2 * D), jnp.float32),
            pltpu.VMEM((2, D, 128), jnp.float32),
            pltpu.SemaphoreType.DMA,
            pltpu.SemaphoreType.DMA,
        ],
        compiler_params=pltpu.CompilerParams(needs_layout_passes=False),
    )
    def k(idxT_hbm, table_hbm, out_hbm, idx_v, rows_v, ot_v, gsem, wsem):
        wid = lax.axis_index("s") * nc + lax.axis_index("c")
        b0 = wid * 128
        iotas = [lax.iota(jnp.int32, _L) + k * _L for k in range(128 // _L)]
        pltpu.sync_copy(idxT_hbm.at[:, pl.ds(b0, 128)], idx_v)
        pltpu.async_copy(table_hbm.at[idx_v.at[0]], rows_v.at[0], gsem)

        @pl.loop(0, S)
        def _(s):
            b = s % 2
            pltpu.make_async_copy(
                table_hbm.at[pl.ds(0, 128)], rows_v.at[b], gsem
            ).wait()

            @pl.when(s + 1 < S)
            def _():
                pltpu.async_copy(
                    table_hbm.at[idx_v.at[s + 1]], rows_v.at[1 - b], gsem
                )

            @pl.when(s > 1)
            def _():
                pltpu.make_async_copy(
                    out_hbm.at[0, :, pl.ds(0, 128)], ot_v.at[b], wsem
                ).wait()

            _transpose_into(rows_v.at[b], ot_v.at[b], 128, D, iotas)
            pltpu.async_copy(ot_v.at[b], out_hbm.at[s, :, pl.ds(b0, 128)], wsem)

        @pl.loop(0, 2)
        def _(i):
            pltpu.make_async_copy(out_hbm.at[0, :, pl.ds(0, 128)], ot_v.at[0], wsem).wait()

    return k


def kernel(inputs, table):
    NB, S = inputs.shape
    V, D = table.shape
    idxT = inputs.T.astype(jnp.int32)
    tableT = table.T
    n_win = V // 128
    tail = jnp.pad(table[n_win * 128:, :], ((0, 128 - (V - n_win * 128)), (0, D)))
    tpad = _make_fmt(V, D)(tableT, tail)
    outc = _make_gather(S, NB, D)(idxT, tpad)
    emb = lax.transpose(outc, (2, 0, 1))
    return emb, inputs


# speedup vs baseline: 2.6832x; 2.6832x over previous
import functools

import jax
import jax.numpy as jnp
from jax import lax
from jax.experimental import pallas as pl
from jax.experimental.pallas import tpu as pltpu
from jax.experimental.pallas import tpu_sc as plsc


def _make_gather(B, D, C):
    info = plsc.get_sparse_core_info()
    nc, ns = info.num_cores, info.num_subcores
    nw = nc * ns
    n_per_w = B // nw
    n_chunks = n_per_w // C
    mesh = plsc.VectorSubcoreMesh(core_axis_name="c", subcore_axis_name="s")

    @functools.partial(
        pl.kernel,
        out_type=jax.ShapeDtypeStruct((B, 2 * D), jnp.float32),
        mesh=mesh,
        scratch_types=[
            pltpu.VMEM((C,), jnp.int32),
            pltpu.VMEM((C,), jnp.int32),
            pltpu.VMEM((C, 2 * D), jnp.float32),
            pltpu.VMEM((C, 2 * D), jnp.float32),
            pltpu.SemaphoreType.DMA,
            pltpu.SemaphoreType.DMA,
            pltpu.SemaphoreType.DMA,
        ],
        compiler_params=pltpu.CompilerParams(use_tc_tiling_on_sc=True),
    )
    def k(idx_hbm, table_hbm, out_hbm, idx_a, idx_b, rows_a, rows_b, isem, gsem, wsem):
        wid = lax.axis_index("s") * nc + lax.axis_index("c")
        base0 = wid * n_per_w
        bufs = ((idx_a, rows_a), (idx_b, rows_b))

        pltpu.sync_copy(idx_hbm.at[pl.ds(base0, C)], idx_a)
        pltpu.async_copy(table_hbm.at[idx_a], rows_a, gsem)
        pltpu.async_copy(idx_hbm.at[pl.ds(base0 + C, C)], idx_b, isem)

        @pl.loop(0, n_chunks, step=2)
        def _(j):
            for b in (0, 1):
                jj = j + b
                idx_c, rows_c = bufs[b]
                idx_n, rows_n = bufs[1 - b]
                pltpu.make_async_copy(
                    table_hbm.at[pl.ds(0, C)], rows_c, gsem
                ).wait()
                pltpu.async_copy(
                    rows_c, out_hbm.at[pl.ds(base0 + jj * C, C)], wsem
                )

                @pl.when(jj + 1 < n_chunks)
                def _():
                    pltpu.make_async_copy(
                        idx_hbm.at[pl.ds(0, C)], idx_n, isem
                    ).wait()

                    @pl.when(jj > 0)
                    def _():
                        pltpu.make_async_copy(
                            out_hbm.at[pl.ds(0, C)], rows_n, wsem
                        ).wait()

                    pltpu.async_copy(table_hbm.at[idx_n], rows_n, gsem)

                    @pl.when(jj + 2 < n_chunks)
                    def _():
                        pltpu.async_copy(
                            idx_hbm.at[pl.ds(base0 + (jj + 2) * C, C)], idx_c, isem
                        )

        pltpu.make_async_copy(out_hbm.at[pl.ds(0, C)], rows_a, wsem).wait()
        pltpu.make_async_copy(out_hbm.at[pl.ds(0, C)], rows_a, wsem).wait()

    return k


def kernel(inputs, table):
    D = table.shape[1]
    B = inputs.shape[0] * inputs.shape[1]
    idx = inputs.reshape(B).astype(jnp.int32)
    tpad = jnp.pad(table, ((0, 0), (0, D)))
    out = _make_gather(B, D, 400)(idx, tpad)
    return out[:, :D].reshape(inputs.shape + (D,)), inputs


# speedup vs baseline: 2.6879x; 1.0017x over previous
import functools

import jax
import jax.numpy as jnp
from jax import lax
from jax.experimental import pallas as pl
from jax.experimental.pallas import tpu as pltpu
from jax.experimental.pallas import tpu_sc as plsc


def _make_gather(B, D, C):
    info = plsc.get_sparse_core_info()
    nc, ns = info.num_cores, info.num_subcores
    nw = nc * ns
    n_per_w = B // nw
    n_chunks = n_per_w // C
    mesh = plsc.VectorSubcoreMesh(core_axis_name="c", subcore_axis_name="s")

    @functools.partial(
        pl.kernel,
        out_type=jax.ShapeDtypeStruct((B, 2 * D), jnp.float32),
        mesh=mesh,
        scratch_types=[
            pltpu.VMEM((C,), jnp.int32),
            pltpu.VMEM((C,), jnp.int32),
            pltpu.VMEM((C, 2 * D), jnp.float32),
            pltpu.VMEM((C, 2 * D), jnp.float32),
            pltpu.SemaphoreType.DMA,
            pltpu.SemaphoreType.DMA,
            pltpu.SemaphoreType.DMA,
        ],
        compiler_params=pltpu.CompilerParams(
            use_tc_tiling_on_sc=True,
            disable_bounds_checks=True,
        ),
    )
    def k(idx_hbm, table_hbm, out_hbm, idx_a, idx_b, rows_a, rows_b, isem, gsem, wsem):
        wid = lax.axis_index("s") * nc + lax.axis_index("c")
        base0 = wid * n_per_w
        bufs = ((idx_a, rows_a), (idx_b, rows_b))

        pltpu.sync_copy(idx_hbm.at[pl.ds(base0, C)], idx_a)
        pltpu.async_copy(table_hbm.at[idx_a], rows_a, gsem)
        pltpu.async_copy(idx_hbm.at[pl.ds(base0 + C, C)], idx_b, isem)

        @pl.loop(0, n_chunks, step=2)
        def _(j):
            for b in (0, 1):
                jj = j + b
                idx_c, rows_c = bufs[b]
                idx_n, rows_n = bufs[1 - b]
                pltpu.make_async_copy(
                    table_hbm.at[pl.ds(0, C)], rows_c, gsem
                ).wait()
                pltpu.async_copy(
                    rows_c, out_hbm.at[pl.ds(base0 + jj * C, C)], wsem
                )

                @pl.when(jj + 1 < n_chunks)
                def _():
                    pltpu.make_async_copy(
                        idx_hbm.at[pl.ds(0, C)], idx_n, isem
                    ).wait()

                    @pl.when(jj > 0)
                    def _():
                        pltpu.make_async_copy(
                            out_hbm.at[pl.ds(0, C)], rows_n, wsem
                        ).wait()

                    pltpu.async_copy(table_hbm.at[idx_n], rows_n, gsem)

                    @pl.when(jj + 2 < n_chunks)
                    def _():
                        pltpu.async_copy(
                            idx_hbm.at[pl.ds(base0 + (jj + 2) * C, C)], idx_c, isem
                        )

        pltpu.make_async_copy(out_hbm.at[pl.ds(0, C)], rows_a, wsem).wait()
        pltpu.make_async_copy(out_hbm.at[pl.ds(0, C)], rows_a, wsem).wait()

    return k


def kernel(inputs, table):
    D = table.shape[1]
    B = inputs.shape[0] * inputs.shape[1]
    idx = inputs.reshape(B).astype(jnp.int32)
    tpad = jnp.pad(table, ((0, 0), (0, D)))
    out = _make_gather(B, D, 400)(idx, tpad)
    return out[:, :D].reshape(inputs.shape + (D,)), inputs
